# baseline (speedup 1.0000x reference)
import functools

import jax
import jax.numpy as jnp
from jax import lax
from jax.experimental import pallas as pl
from jax.experimental.pallas import tpu as pltpu
from jax.experimental.pallas import tpu_sc as plsc

NUM_WORKERS = 32
BLK = 128
NBUF = 4
NTILE = 2


def _make_gather(batch: int, seq: int, vocab: int, dim: int):
  assert batch == NUM_WORKERS * BLK
  n_ct = dim // 8
  mesh = plsc.VectorSubcoreMesh(core_axis_name="c", subcore_axis_name="s")

  @functools.partial(
      pl.kernel,
      mesh=mesh,
      out_type=jax.ShapeDtypeStruct((seq, n_ct, NUM_WORKERS, 8, BLK),
                                    jnp.float32),
      scratch_types=[
          pltpu.VMEM((BLK, seq), jnp.int32),
          pltpu.VMEM((seq, BLK), jnp.int32),
          pltpu.VMEM((NBUF, BLK, dim), jnp.float32),
          pltpu.VMEM((NTILE, n_ct, 8, BLK), jnp.float32),
          pltpu.SemaphoreType.DMA((NBUF,)),
          pltpu.SemaphoreType.DMA((NTILE,)),
      ],
      compiler_params=pltpu.CompilerParams(use_tc_tiling_on_sc=False,
                                           needs_layout_passes=False),
  )
  def gather_kernel(ids_hbm, table_hbm, out_hbm, idsv, idst, rows_v, tile_v,
                    gsem, wsem):
    c = lax.axis_index("c")
    s_ax = lax.axis_index("s")
    wid = s_ax * 2 + c
    pltpu.sync_copy(ids_hbm.at[pl.ds(wid * BLK, BLK)], idsv)

    lane = lax.iota(jnp.int32, 16)

    def tr_ids(s, carry):
      col = jnp.full((16,), s, jnp.int32)
      for i0 in range(0, BLK, 16):
        v = plsc.load_gather(idsv, [lane + i0, col])
        idst[s, pl.ds(i0, 16)] = v
      return carry

    lax.fori_loop(0, seq, tr_ids, 0)

    def gather(s, slot):
      pltpu.async_copy(table_hbm.at[idst.at[s]], rows_v.at[slot],
                       gsem.at[slot])

    def wait_gather(s, slot):
      pltpu.make_async_copy(table_hbm.at[idst.at[s]], rows_v.at[slot],
                            gsem.at[slot]).wait()

    def transpose(slot, t):
      def tr_col(cc, carry):
        col = jnp.full((16,), cc, jnp.int32)
        ct = cc // 8
        cs = lax.rem(cc, 8)
        for i0 in range(0, BLK, 16):
          v = plsc.load_gather(rows_v.at[slot], [lane + i0, col])
          tile_v[t, ct, cs, pl.ds(i0, 16)] = v
        return carry

      lax.fori_loop(0, dim, tr_col, 0)

    def writeback(s, t):
      pltpu.async_copy(tile_v.at[t], out_hbm.at[s, :, wid], wsem.at[t])

    def wait_writeback(s, t):
      pltpu.make_async_copy(tile_v.at[t], out_hbm.at[s, :, wid],
                            wsem.at[t]).wait()

    for s in range(NBUF):
      gather(s, s)

    for s in range(NTILE):
      wait_gather(s, s % NBUF)
      transpose(s % NBUF, s % NTILE)
      writeback(s, s % NTILE)
      gather(s + NBUF, s % NBUF)

    def body(s, carry):
      slot = lax.rem(s, NBUF)
      t = lax.rem(s, NTILE)
      wait_writeback(s - NTILE, t)
      wait_gather(s, slot)
      transpose(slot, t)
      writeback(s, t)
      gather(s + NBUF, slot)
      return carry

    lax.fori_loop(NTILE, seq - NBUF, body, 0)

    for k in range(NBUF):
      s = seq - NBUF + k
      slot = s % NBUF
      t = s % NTILE
      wait_writeback(s - NTILE, t)
      wait_gather(s, slot)
      transpose(slot, t)
      writeback(s, t)

    for k in range(NTILE):
      s = seq - NTILE + k
      wait_writeback(s, s % NTILE)

  return gather_kernel


def kernel(input_ids, attention_mask, emb_weight):
  batch, seq = input_ids.shape
  vocab, dim = emb_weight.shape
  table_wide = jax.lax.optimization_barrier(
      emb_weight.reshape(vocab // 2, 2 * dim))
  table = table_wide.reshape(vocab, dim)
  ids = input_ids.astype(jnp.int32)
  out5d = _make_gather(batch, seq, vocab, dim)(ids, table)
  out = out5d.transpose(2, 4, 0, 1, 3).reshape(batch, seq, dim)
  return out, attention_mask


# speedup vs baseline: 1.8152x; 1.8152x over previous
import functools

import jax
import jax.numpy as jnp
from jax import lax
from jax.experimental import pallas as pl
from jax.experimental.pallas import tpu as pltpu
from jax.experimental.pallas import tpu_sc as plsc

NUM_WORKERS = 32
CHUNK = 128
NBUF = 4


def _make_gather(n_ids: int, vocab: int, padded_dim: int):
  n_per_w = n_ids // NUM_WORKERS
  n_chunks = n_per_w // CHUNK
  mesh = plsc.VectorSubcoreMesh(core_axis_name="c", subcore_axis_name="s")

  dim = 64

  @functools.partial(
      pl.kernel,
      mesh=mesh,
      out_type=jax.ShapeDtypeStruct((n_ids, padded_dim), jnp.float32),
      scratch_types=[
          pltpu.VMEM((n_chunks, CHUNK), jnp.int32),
          pltpu.VMEM((NBUF, CHUNK, padded_dim), jnp.float32),
          pltpu.SemaphoreType.DMA((NBUF,)),
      ],
      compiler_params=pltpu.CompilerParams(use_tc_tiling_on_sc=False,
                                           needs_layout_passes=False),
  )
  def gather_kernel(ids_hbm, table_hbm, out_hbm, idx_v, rows_v, gsem):
    c = lax.axis_index("c")
    s = lax.axis_index("s")
    wid = s * 2 + c
    base = wid * n_per_w
    pltpu.sync_copy(ids_hbm.at[wid], idx_v)

    for b in range(NBUF):
      pltpu.async_copy(table_hbm.at[idx_v.at[b]], rows_v.at[b], gsem.at[b])

    def body(j, carry):
      slot = lax.rem(j, NBUF)
      pltpu.make_async_copy(
          table_hbm.at[idx_v.at[j]], rows_v.at[slot], gsem.at[slot]).wait()
      pltpu.sync_copy(
          rows_v.at[slot], out_hbm.at[pl.ds(base + j * CHUNK, CHUNK)])
      nxt = j + NBUF
      pltpu.async_copy(table_hbm.at[idx_v.at[nxt]], rows_v.at[slot],
                       gsem.at[slot])
      return carry

    lax.fori_loop(0, n_chunks - NBUF, body, 0)

    for b in range(NBUF):
      j = n_chunks - NBUF + b
      slot = j % NBUF
      pltpu.make_async_copy(
          table_hbm.at[idx_v.at[j]], rows_v.at[slot], gsem.at[slot]).wait()
      pltpu.sync_copy(
          rows_v.at[slot], out_hbm.at[pl.ds(base + j * CHUNK, CHUNK)])

  return gather_kernel


def kernel(input_ids, attention_mask, emb_weight):
  batch, seq = input_ids.shape
  vocab, dim = emb_weight.shape
  n_ids = batch * seq
  table = jax.lax.optimization_barrier(
      jnp.pad(emb_weight, ((0, 0), (0, 128 - dim))))
  ids = input_ids.reshape(NUM_WORKERS, n_ids // (NUM_WORKERS * CHUNK), CHUNK)
  ids = ids.astype(jnp.int32)
  out128 = _make_gather(n_ids, vocab, 128)(ids, table)
  return out128[:, :dim].reshape(batch, seq, dim), attention_mask


# speedup vs baseline: 1.9751x; 1.0881x over previous
import functools

import jax
import jax.numpy as jnp
from jax import lax
from jax.experimental import pallas as pl
from jax.experimental.pallas import tpu as pltpu
from jax.experimental.pallas import tpu_sc as plsc

NUM_WORKERS = 32
CHUNK = 128
NBUF = 8
AHEAD = 4


def _make_gather(n_ids: int, vocab: int, padded_dim: int):
  n_per_w = n_ids // NUM_WORKERS
  n_chunks = n_per_w // CHUNK
  mesh = plsc.VectorSubcoreMesh(core_axis_name="c", subcore_axis_name="s")

  dim = 64

  @functools.partial(
      pl.kernel,
      mesh=mesh,
      out_type=jax.ShapeDtypeStruct((n_ids, padded_dim), jnp.float32),
      scratch_types=[
          pltpu.VMEM((n_chunks, CHUNK), jnp.int32),
          pltpu.VMEM((NBUF, CHUNK, dim), jnp.float32),
          pltpu.SemaphoreType.DMA((NBUF,)),
          pltpu.SemaphoreType.DMA((NBUF,)),
      ],
      compiler_params=pltpu.CompilerParams(use_tc_tiling_on_sc=False,
                                           needs_layout_passes=False),
  )
  def gather_kernel(ids_hbm, table_hbm, out_hbm, idx_v, rows_v, gsem, wsem):
    c = lax.axis_index("c")
    s = lax.axis_index("s")
    wid = s * 2 + c
    base = wid * n_per_w
    pltpu.sync_copy(ids_hbm.at[wid], idx_v)

    def gather(j, slot):
      pltpu.async_copy(table_hbm.at[idx_v.at[j]], rows_v.at[slot],
                       gsem.at[slot])

    def wait_gather(j, slot):
      pltpu.make_async_copy(table_hbm.at[idx_v.at[j]], rows_v.at[slot],
                            gsem.at[slot]).wait()

    def put(j, slot):
      pltpu.async_copy(
          rows_v.at[slot],
          out_hbm.at[pl.ds(base + j * CHUNK, CHUNK), pl.ds(0, dim)],
          wsem.at[slot])

    def wait_put(j, slot):
      pltpu.make_async_copy(
          rows_v.at[slot],
          out_hbm.at[pl.ds(base + j * CHUNK, CHUNK), pl.ds(0, dim)],
          wsem.at[slot]).wait()

    for b in range(AHEAD):
      gather(b, b)

    for j in range(NBUF - AHEAD):
      wait_gather(j, j)
      put(j, j)
      gather(j + AHEAD, (j + AHEAD) % NBUF)

    def body(j, carry):
      slot = lax.rem(j, NBUF)
      wait_gather(j, slot)
      put(j, slot)
      nslot = lax.rem(j + AHEAD, NBUF)
      wait_put(j + AHEAD - NBUF, nslot)
      gather(j + AHEAD, nslot)
      return carry

    lax.fori_loop(NBUF - AHEAD, n_chunks - AHEAD, body, 0)

    for k in range(AHEAD):
      j = n_chunks - AHEAD + k
      slot = j % NBUF
      wait_gather(j, slot)
      put(j, slot)
    for k in range(NBUF):
      j = n_chunks - NBUF + k
      wait_put(j, j % NBUF)

  return gather_kernel


def kernel(input_ids, attention_mask, emb_weight):
  batch, seq = input_ids.shape
  vocab, dim = emb_weight.shape
  n_ids = batch * seq
  table_wide = jax.lax.optimization_barrier(emb_weight.reshape(vocab // 2, 2 * dim))
  table = table_wide.reshape(vocab, dim)
  ids = input_ids.reshape(NUM_WORKERS, n_ids // (NUM_WORKERS * CHUNK), CHUNK)
  ids = ids.astype(jnp.int32)
  out128 = _make_gather(n_ids, vocab, 128)(ids, table)
  return out128[:, :dim].reshape(batch, seq, dim), attention_mask
